# baseline (device time: 579471 ns/iter reference)
import jax
import jax.numpy as jnp
from jax import lax
from jax.experimental import pallas as pl
from jax.experimental.pallas import tpu as pltpu

T = 1024
HALF = 16384
CHUNK = 2048
NC = HALF // CHUNK


def kernel(x, W):
    logits = jnp.dot(x, W, preferred_element_type=jnp.float32)
    m_l = jnp.max(logits, axis=1, keepdims=True)
    s_l = jnp.sum(jnp.exp(logits - m_l), axis=1, keepdims=True)
    stats = jnp.concatenate([m_l, s_l], axis=1)

    def body(logits_ref, stats_ref, out_ref, recv_hbm,
             stats_recv, tile, norm, normbf, rbf, pnorm,
             load_sems, store_sems, pload_sem, pstore_sems,
             st_send_sem, st_recv_sem, send_sems, recv_sems):
        xi = lax.axis_index("x")
        yi = lax.axis_index("y")
        zi = lax.axis_index("z")
        partner = (xi, yi, 1 - zi)

        barrier = pltpu.get_barrier_semaphore()
        pl.semaphore_signal(barrier, inc=1, device_id=partner,
                            device_id_type=pl.DeviceIdType.MESH)
        pl.semaphore_wait(barrier, 1)

        st_rdma = pltpu.make_async_remote_copy(
            src_ref=stats_ref, dst_ref=stats_recv,
            send_sem=st_send_sem, recv_sem=st_recv_sem,
            device_id=partner, device_id_type=pl.DeviceIdType.MESH)
        st_rdma.start()
        st_rdma.wait()

        m_mine = stats_ref[:, 0:1]
        s_mine = stats_ref[:, 1:2]
        m_peer = stats_recv[:, 0:1]
        s_peer = stats_recv[:, 1:2]
        m = jnp.maximum(m_mine, m_peer)
        s = s_mine * jnp.exp(m_mine - m) + s_peer * jnp.exp(m_peer - m)
        inv_s = 1.0 / s

        col0 = zi * HALF
        pcol0 = (1 - zi) * HALF

        loads, stores, sends, pstores = [], [], [], []

        def process_recv(r):
            rb = r % 2
            sends[r].wait_recv()
            pld = pltpu.make_async_copy(
                recv_hbm.at[:, pl.ds(r * CHUNK, CHUNK)], rbf, pload_sem)
            pld.start()
            pld.wait()
            if r >= 2:
                pstores[r - 2].wait()
            pnorm[rb] = rbf[...].astype(jnp.float32)
            pst = pltpu.make_async_copy(
                pnorm.at[rb], out_ref.at[:, pl.ds(pcol0 + r * CHUNK, CHUNK)],
                pstore_sems.at[rb])
            pst.start()
            pstores.append(pst)

        ld0 = pltpu.make_async_copy(
            logits_ref.at[:, pl.ds(0, CHUNK)], tile.at[0], load_sems.at[0])
        ld0.start()
        loads.append(ld0)
        for c in range(NC):
            b = c % 2
            loads[c].wait()
            if c + 1 < NC:
                nb = (c + 1) % 2
                ldn = pltpu.make_async_copy(
                    logits_ref.at[:, pl.ds((c + 1) * CHUNK, CHUNK)],
                    tile.at[nb], load_sems.at[nb])
                ldn.start()
                loads.append(ldn)
            if c >= 2:
                sends[c - 2].wait_send()
            if c >= 1:
                stores[c - 1].wait()
            v = jnp.exp(tile[b] - m) * inv_s
            norm[...] = v
            normbf[b] = v.astype(jnp.bfloat16)
            stc = pltpu.make_async_copy(
                norm, out_ref.at[:, pl.ds(col0 + c * CHUNK, CHUNK)],
                store_sems.at[b])
            stc.start()
            stores.append(stc)
            rdma = pltpu.make_async_remote_copy(
                src_ref=normbf.at[b],
                dst_ref=recv_hbm.at[:, pl.ds(c * CHUNK, CHUNK)],
                send_sem=send_sems.at[c], recv_sem=recv_sems.at[c],
                device_id=partner, device_id_type=pl.DeviceIdType.MESH)
            rdma.start()
            sends.append(rdma)
            if c >= 1:
                process_recv(c - 1)
        process_recv(NC - 1)

        for c in range(max(NC - 2, 0), NC):
            sends[c].wait_send()
            pstores[c].wait()
        stores[NC - 1].wait()

    out, _ = pl.pallas_call(
        body,
        out_shape=(
            jax.ShapeDtypeStruct((T, 2 * HALF), jnp.float32),
            jax.ShapeDtypeStruct((T, HALF), jnp.bfloat16),
        ),
        in_specs=[
            pl.BlockSpec(memory_space=pl.ANY),
            pl.BlockSpec(memory_space=pltpu.VMEM),
        ],
        out_specs=(
            pl.BlockSpec(memory_space=pl.ANY),
            pl.BlockSpec(memory_space=pl.ANY),
        ),
        scratch_shapes=[
            pltpu.VMEM((T, 2), jnp.float32),
            pltpu.VMEM((2, T, CHUNK), jnp.float32),
            pltpu.VMEM((T, CHUNK), jnp.float32),
            pltpu.VMEM((2, T, CHUNK), jnp.bfloat16),
            pltpu.VMEM((T, CHUNK), jnp.bfloat16),
            pltpu.VMEM((2, T, CHUNK), jnp.float32),
            pltpu.SemaphoreType.DMA((2,)),
            pltpu.SemaphoreType.DMA((2,)),
            pltpu.SemaphoreType.DMA,
            pltpu.SemaphoreType.DMA((2,)),
            pltpu.SemaphoreType.DMA,
            pltpu.SemaphoreType.DMA,
            pltpu.SemaphoreType.DMA((NC,)),
            pltpu.SemaphoreType.DMA((NC,)),
        ],
        compiler_params=pltpu.CompilerParams(
            collective_id=0, vmem_limit_bytes=64 * 1024 * 1024),
    )(logits, stats)
    return out


# device time: 538427 ns/iter; 1.0762x vs baseline; 1.0762x over previous
import jax
import jax.numpy as jnp
from jax import lax
from jax.experimental import pallas as pl
from jax.experimental.pallas import tpu as pltpu

T = 1024
D = 2048
HALF = 16384
CH = 512
NC = HALF // CH
CH2 = 2048
NC2 = HALF // CH2
LMAX = 8.0


def kernel(x, W):
    xb = x.astype(jnp.bfloat16)

    def body(xb_ref, w_hbm, out_ref, recv_hbm, ebf_hbm,
             stats_recv, wbuf, ebf, mbf, rbf, mnorm, pnorm, s_ref,
             wload_sems, estore_sems, send_sems, recv_sems,
             st_send_sem, st_recv_sem, mload_sems, pload_sems,
             mstore_sem, pstore_sem):
        xi = lax.axis_index("x")
        yi = lax.axis_index("y")
        zi = lax.axis_index("z")
        partner = (xi, yi, 1 - zi)

        barrier = pltpu.get_barrier_semaphore()
        pl.semaphore_signal(barrier, inc=1, device_id=partner,
                            device_id_type=pl.DeviceIdType.MESH)
        pl.semaphore_wait(barrier, 1)

        col0 = zi * HALF
        pcol0 = (1 - zi) * HALF

        wloads, estores, sends = [], [], []
        ld0 = pltpu.make_async_copy(
            w_hbm.at[:, pl.ds(0, CH)], wbuf.at[0], wload_sems.at[0])
        ld0.start()
        wloads.append(ld0)
        s_ref[...] = jnp.zeros((T, 1), jnp.float32)
        for c in range(NC):
            b = c % 2
            wloads[c].wait()
            if c + 1 < NC:
                nb = (c + 1) % 2
                ldn = pltpu.make_async_copy(
                    w_hbm.at[:, pl.ds((c + 1) * CH, CH)], wbuf.at[nb],
                    wload_sems.at[nb])
                ldn.start()
                wloads.append(ldn)
            wb = wbuf[b].astype(jnp.bfloat16)
            l = jnp.dot(xb_ref[...], wb, preferred_element_type=jnp.float32)
            e = jnp.exp(l - LMAX)
            s_ref[...] += jnp.sum(e, axis=1, keepdims=True)
            if c >= 2:
                sends[c - 2].wait_send()
                estores[c - 2].wait()
            ebf[b] = e.astype(jnp.bfloat16)
            est = pltpu.make_async_copy(
                ebf.at[b], ebf_hbm.at[:, pl.ds(c * CH, CH)],
                estore_sems.at[b])
            est.start()
            estores.append(est)
            rdma = pltpu.make_async_remote_copy(
                src_ref=ebf.at[b],
                dst_ref=recv_hbm.at[:, pl.ds(c * CH, CH)],
                send_sem=send_sems.at[c], recv_sem=recv_sems.at[c],
                device_id=partner, device_id_type=pl.DeviceIdType.MESH)
            rdma.start()
            sends.append(rdma)

        st_rdma = pltpu.make_async_remote_copy(
            src_ref=s_ref, dst_ref=stats_recv,
            send_sem=st_send_sem, recv_sem=st_recv_sem,
            device_id=partner, device_id_type=pl.DeviceIdType.MESH)
        st_rdma.start()
        st_rdma.wait()
        inv = 1.0 / (s_ref[...] + stats_recv[...])

        estores[NC - 2].wait()
        estores[NC - 1].wait()

        def issue_loads(r):
            rb = r % 2
            mld = pltpu.make_async_copy(
                ebf_hbm.at[:, pl.ds(r * CH2, CH2)], mbf.at[rb],
                mload_sems.at[rb])
            mld.start()
            for q in range(CH2 // CH):
                sends[r * (CH2 // CH) + q].wait_recv()
            pld = pltpu.make_async_copy(
                recv_hbm.at[:, pl.ds(r * CH2, CH2)], rbf.at[rb],
                pload_sems.at[rb])
            pld.start()
            return mld, pld

        mstores, pstores = [], []
        loads2 = [issue_loads(0)]
        for r in range(NC2):
            rb = r % 2
            mld, pld = loads2[r]
            mld.wait()
            pld.wait()
            if r + 1 < NC2:
                loads2.append(issue_loads(r + 1))
            if r >= 1:
                mstores[r - 1].wait()
                pstores[r - 1].wait()
            mnorm[...] = mbf[rb].astype(jnp.float32) * inv
            pnorm[...] = rbf[rb].astype(jnp.float32) * inv
            mst = pltpu.make_async_copy(
                mnorm, out_ref.at[:, pl.ds(col0 + r * CH2, CH2)],
                mstore_sem)
            mst.start()
            mstores.append(mst)
            pst = pltpu.make_async_copy(
                pnorm, out_ref.at[:, pl.ds(pcol0 + r * CH2, CH2)],
                pstore_sem)
            pst.start()
            pstores.append(pst)
        mstores[NC2 - 1].wait()
        pstores[NC2 - 1].wait()
        for c in range(max(NC - 2, 0), NC):
            sends[c].wait_send()

    out, _, _ = pl.pallas_call(
        body,
        out_shape=(
            jax.ShapeDtypeStruct((T, 2 * HALF), jnp.float32),
            jax.ShapeDtypeStruct((T, HALF), jnp.bfloat16),
            jax.ShapeDtypeStruct((T, HALF), jnp.bfloat16),
        ),
        in_specs=[
            pl.BlockSpec(memory_space=pltpu.VMEM),
            pl.BlockSpec(memory_space=pl.ANY),
        ],
        out_specs=(
            pl.BlockSpec(memory_space=pl.ANY),
            pl.BlockSpec(memory_space=pl.ANY),
            pl.BlockSpec(memory_space=pl.ANY),
        ),
        scratch_shapes=[
            pltpu.VMEM((T, 1), jnp.float32),
            pltpu.VMEM((2, D, CH), jnp.float32),
            pltpu.VMEM((2, T, CH), jnp.bfloat16),
            pltpu.VMEM((2, T, CH2), jnp.bfloat16),
            pltpu.VMEM((2, T, CH2), jnp.bfloat16),
            pltpu.VMEM((T, CH2), jnp.float32),
            pltpu.VMEM((T, CH2), jnp.float32),
            pltpu.VMEM((T, 1), jnp.float32),
            pltpu.SemaphoreType.DMA((2,)),
            pltpu.SemaphoreType.DMA((2,)),
            pltpu.SemaphoreType.DMA((NC,)),
            pltpu.SemaphoreType.DMA((NC,)),
            pltpu.SemaphoreType.DMA,
            pltpu.SemaphoreType.DMA,
            pltpu.SemaphoreType.DMA((2,)),
            pltpu.SemaphoreType.DMA((2,)),
            pltpu.SemaphoreType.DMA,
            pltpu.SemaphoreType.DMA,
        ],
        compiler_params=pltpu.CompilerParams(
            collective_id=0, vmem_limit_bytes=48 * 1024 * 1024),
    )(xb, W)
    return out


# device time: 536884 ns/iter; 1.0793x vs baseline; 1.0029x over previous
import jax
import jax.numpy as jnp
from jax import lax
from jax.experimental import pallas as pl
from jax.experimental.pallas import tpu as pltpu

T = 1024
D = 2048
HALF = 16384
CH = 512
NC = HALF // CH
CH2 = 1024
NC2 = HALF // CH2
R = CH2 // CH
LMAX = 8.0


def kernel(x, W):
    xb = x.astype(jnp.bfloat16)

    def body(xb_ref, w_hbm, out_ref, recv_hbm,
             stats_recv, wbuf, ebf_all, rbf, mnorm, pnorm, s_ref,
             wload_sems, send_sems, recv_sems,
             st_send_sem, st_recv_sem, pload_sem,
             mstore_sem, pstore_sem):
        xi = lax.axis_index("x")
        yi = lax.axis_index("y")
        zi = lax.axis_index("z")
        partner = (xi, yi, 1 - zi)

        barrier = pltpu.get_barrier_semaphore()
        pl.semaphore_signal(barrier, inc=1, device_id=partner,
                            device_id_type=pl.DeviceIdType.MESH)
        pl.semaphore_wait(barrier, 1)

        col0 = zi * HALF
        pcol0 = (1 - zi) * HALF

        wloads, sends = [], []
        ld0 = pltpu.make_async_copy(
            w_hbm.at[:, pl.ds(0, CH)], wbuf.at[0], wload_sems.at[0])
        ld0.start()
        wloads.append(ld0)
        s_ref[...] = jnp.zeros((T, 1), jnp.float32)
        for c in range(NC):
            b = c % 2
            wloads[c].wait()
            if c + 1 < NC:
                nb = (c + 1) % 2
                ldn = pltpu.make_async_copy(
                    w_hbm.at[:, pl.ds((c + 1) * CH, CH)], wbuf.at[nb],
                    wload_sems.at[nb])
                ldn.start()
                wloads.append(ldn)
            wb = wbuf[b].astype(jnp.bfloat16)
            l = jnp.dot(xb_ref[...], wb, preferred_element_type=jnp.float32)
            e = jnp.exp(l - LMAX)
            s_ref[...] += jnp.sum(e, axis=1, keepdims=True)
            ebf_all[c] = e.astype(jnp.bfloat16)
            rdma = pltpu.make_async_remote_copy(
                src_ref=ebf_all.at[c],
                dst_ref=recv_hbm.at[:, pl.ds(c * CH, CH)],
                send_sem=send_sems.at[c], recv_sem=recv_sems.at[c],
                device_id=partner, device_id_type=pl.DeviceIdType.MESH)
            rdma.start()
            sends.append(rdma)

        st_rdma = pltpu.make_async_remote_copy(
            src_ref=s_ref, dst_ref=stats_recv,
            send_sem=st_send_sem, recv_sem=st_recv_sem,
            device_id=partner, device_id_type=pl.DeviceIdType.MESH)
        st_rdma.start()
        st_rdma.wait()
        inv = 1.0 / (s_ref[...] + stats_recv[...])

        mstores, pstores = [], []
        for r in range(NC2):
            for q in range(R):
                sends[r * R + q].wait_recv()
            pld = pltpu.make_async_copy(
                recv_hbm.at[:, pl.ds(r * CH2, CH2)], rbf, pload_sem)
            pld.start()
            pld.wait()
            if r >= 1:
                mstores[r - 1].wait()
                pstores[r - 1].wait()
            for q in range(R):
                mnorm[:, q * CH:(q + 1) * CH] = (
                    ebf_all[r * R + q].astype(jnp.float32) * inv)
            pnorm[...] = rbf[...].astype(jnp.float32) * inv
            mst = pltpu.make_async_copy(
                mnorm, out_ref.at[:, pl.ds(col0 + r * CH2, CH2)],
                mstore_sem)
            mst.start()
            mstores.append(mst)
            pst = pltpu.make_async_copy(
                pnorm, out_ref.at[:, pl.ds(pcol0 + r * CH2, CH2)],
                pstore_sem)
            pst.start()
            pstores.append(pst)
        mstores[NC2 - 1].wait()
        pstores[NC2 - 1].wait()
        for c in range(NC):
            sends[c].wait_send()

    out, _ = pl.pallas_call(
        body,
        out_shape=(
            jax.ShapeDtypeStruct((T, 2 * HALF), jnp.float32),
            jax.ShapeDtypeStruct((T, HALF), jnp.bfloat16),
        ),
        in_specs=[
            pl.BlockSpec(memory_space=pltpu.VMEM),
            pl.BlockSpec(memory_space=pl.ANY),
        ],
        out_specs=(
            pl.BlockSpec(memory_space=pl.ANY),
            pl.BlockSpec(memory_space=pl.ANY),
        ),
        scratch_shapes=[
            pltpu.VMEM((T, 1), jnp.float32),
            pltpu.VMEM((2, D, CH), jnp.float32),
            pltpu.VMEM((NC, T, CH), jnp.bfloat16),
            pltpu.VMEM((T, CH2), jnp.bfloat16),
            pltpu.VMEM((T, CH2), jnp.float32),
            pltpu.VMEM((T, CH2), jnp.float32),
            pltpu.VMEM((T, 1), jnp.float32),
            pltpu.SemaphoreType.DMA((2,)),
            pltpu.SemaphoreType.DMA((NC,)),
            pltpu.SemaphoreType.DMA((NC,)),
            pltpu.SemaphoreType.DMA,
            pltpu.SemaphoreType.DMA,
            pltpu.SemaphoreType.DMA,
            pltpu.SemaphoreType.DMA,
            pltpu.SemaphoreType.DMA,
        ],
        compiler_params=pltpu.CompilerParams(
            collective_id=0, vmem_limit_bytes=64 * 1024 * 1024),
    )(xb, W)
    return out


# device time: 535961 ns/iter; 1.0812x vs baseline; 1.0017x over previous
import functools

import jax
import jax.numpy as jnp
from jax import lax
from jax.experimental import pallas as pl
from jax.experimental.pallas import tpu as pltpu

T = 1024
D = 2048
HALF = 16384
CH = 512
NC = HALF // CH
CH2 = 1024
NC2 = HALF // CH2
R = CH2 // CH
LMAX = 8.0


def kernel(x, W):
    xb = x.astype(jnp.bfloat16)

    def body(xb_ref, wblk, out_ref, recv_hbm,
             stats_recv, ebf_all, rbf, mnorm, pnorm, s_ref,
             send_sems, recv_sems, st_send_sem, st_recv_sem,
             pload_sem, mstore_sem, pstore_sem):
        i = pl.program_id(0)
        xi = lax.axis_index("x")
        yi = lax.axis_index("y")
        zi = lax.axis_index("z")
        partner = (xi, yi, 1 - zi)

        @pl.when(i == 0)
        def _():
            barrier = pltpu.get_barrier_semaphore()
            pl.semaphore_signal(barrier, inc=1, device_id=partner,
                                device_id_type=pl.DeviceIdType.MESH)
            pl.semaphore_wait(barrier, 1)
            s_ref[...] = jnp.zeros((T, 1), jnp.float32)

        wb = wblk[...].astype(jnp.bfloat16)
        l = jnp.dot(xb_ref[...], wb, preferred_element_type=jnp.float32)
        e = jnp.exp(l - LMAX)
        s_ref[...] += jnp.sum(e, axis=1, keepdims=True)
        ebf_all[pl.ds(i, 1)] = e.astype(jnp.bfloat16)[None]
        rdma = pltpu.make_async_remote_copy(
            src_ref=ebf_all.at[i],
            dst_ref=recv_hbm.at[:, pl.ds(i * CH, CH)],
            send_sem=send_sems.at[i], recv_sem=recv_sems.at[i],
            device_id=partner, device_id_type=pl.DeviceIdType.MESH)
        rdma.start()

        @pl.when(i == NC - 1)
        def _():
            st_rdma = pltpu.make_async_remote_copy(
                src_ref=s_ref, dst_ref=stats_recv,
                send_sem=st_send_sem, recv_sem=st_recv_sem,
                device_id=partner, device_id_type=pl.DeviceIdType.MESH)
            st_rdma.start()
            st_rdma.wait()
            inv = 1.0 / (s_ref[...] + stats_recv[...])

            def chunk_desc(c):
                return pltpu.make_async_remote_copy(
                    src_ref=ebf_all.at[c],
                    dst_ref=recv_hbm.at[:, pl.ds(c * CH, CH)],
                    send_sem=send_sems.at[c], recv_sem=recv_sems.at[c],
                    device_id=partner, device_id_type=pl.DeviceIdType.MESH)

            mstores, pstores = [], []
            for r in range(NC2):
                for q in range(R):
                    chunk_desc(r * R + q).wait_recv()
                pld = pltpu.make_async_copy(
                    recv_hbm.at[:, pl.ds(r * CH2, CH2)], rbf, pload_sem)
                pld.start()
                pld.wait()
                if r >= 1:
                    mstores[r - 1].wait()
                    pstores[r - 1].wait()
                for q in range(R):
                    mnorm[:, q * CH:(q + 1) * CH] = (
                        ebf_all[r * R + q].astype(jnp.float32) * inv)
                pnorm[...] = rbf[...].astype(jnp.float32) * inv
                mst = pltpu.make_async_copy(
                    mnorm,
                    out_ref.at[:, pl.ds(zi * HALF + r * CH2, CH2)],
                    mstore_sem)
                mst.start()
                mstores.append(mst)
                pst = pltpu.make_async_copy(
                    pnorm,
                    out_ref.at[:, pl.ds((1 - zi) * HALF + r * CH2, CH2)],
                    pstore_sem)
                pst.start()
                pstores.append(pst)
            mstores[NC2 - 1].wait()
            pstores[NC2 - 1].wait()
            for c in range(NC):
                chunk_desc(c).wait_send()

    out, _ = pl.pallas_call(
        body,
        grid=(NC,),
        out_shape=(
            jax.ShapeDtypeStruct((T, 2 * HALF), jnp.float32),
            jax.ShapeDtypeStruct((T, HALF), jnp.bfloat16),
        ),
        in_specs=[
            pl.BlockSpec((T, D), lambda i: (0, 0)),
            pl.BlockSpec((D, CH), lambda i: (0, i)),
        ],
        out_specs=(
            pl.BlockSpec(memory_space=pl.ANY),
            pl.BlockSpec(memory_space=pl.ANY),
        ),
        scratch_shapes=[
            pltpu.VMEM((T, 1), jnp.float32),
            pltpu.VMEM((NC, T, CH), jnp.bfloat16),
            pltpu.VMEM((T, CH2), jnp.bfloat16),
            pltpu.VMEM((T, CH2), jnp.float32),
            pltpu.VMEM((T, CH2), jnp.float32),
            pltpu.VMEM((T, 1), jnp.float32),
            pltpu.SemaphoreType.DMA((NC,)),
            pltpu.SemaphoreType.DMA((NC,)),
            pltpu.SemaphoreType.DMA,
            pltpu.SemaphoreType.DMA,
            pltpu.SemaphoreType.DMA,
            pltpu.SemaphoreType.DMA,
            pltpu.SemaphoreType.DMA,
        ],
        compiler_params=pltpu.CompilerParams(
            collective_id=0,
            dimension_semantics=("arbitrary",),
            vmem_limit_bytes=60 * 1024 * 1024),
    )(xb, W)
    return out
